# baseline (speedup 1.0000x reference)
import functools

import jax
import jax.numpy as jnp
from jax import lax
from jax.experimental import pallas as pl
from jax.experimental.pallas import tpu as pltpu
from jax.experimental.pallas import tpu_sc as plsc

_NC = 2
_NS = 16
_NW = _NC * _NS

_B = 16384 * 5
_D = 64
_CHUNK = 128
_PER_W = _B // _NW
_NCHUNK = _PER_W // _CHUNK


def _make_gather():
  mesh = plsc.VectorSubcoreMesh(core_axis_name="c", subcore_axis_name="s")

  @functools.partial(
      pl.kernel,
      mesh=mesh,
      compiler_params=pltpu.CompilerParams(use_tc_tiling_on_sc=False),
      out_type=jax.ShapeDtypeStruct((_B, _D), jnp.float32),
      scratch_types=[
          pltpu.VMEM((_NCHUNK, _CHUNK), jnp.int32),
          pltpu.VMEM((_CHUNK, _D), jnp.float32),
          pltpu.VMEM((_CHUNK, _D), jnp.float32),
          pltpu.SemaphoreType.DMA,
          pltpu.SemaphoreType.DMA,
      ],
  )
  def gather_k(idx_hbm, table_hbm, out_hbm, idx_v, rows0, rows1, sem0, sem1):
    wid = lax.axis_index("s") * _NC + lax.axis_index("c")
    base = wid * _PER_W
    pltpu.sync_copy(idx_hbm.at[wid], idx_v)

    rows = (rows0, rows1)
    sems = (sem0, sem1)
    pltpu.async_copy(table_hbm.at[idx_v.at[0]], rows0, sem0)
    for j in range(_NCHUNK):
      if j + 1 < _NCHUNK:
        nxt = (j + 1) % 2
        pltpu.async_copy(table_hbm.at[idx_v.at[j + 1]], rows[nxt], sems[nxt])
      pltpu.make_async_copy(
          table_hbm.at[idx_v.at[j]], rows[j % 2], sems[j % 2]
      ).wait()
      pltpu.sync_copy(rows[j % 2], out_hbm.at[pl.ds(base + j * _CHUNK, _CHUNK)])

  return gather_k


_gather = _make_gather()


def _mm_body(e_ref, w_ref, b_ref, o_ref):
  o_ref[...] = (
      jnp.dot(e_ref[...], w_ref[...], preferred_element_type=jnp.float32)
      + b_ref[...]
  )


def _matmul(e, w_pad, b_pad):
  n = e.shape[0]
  blk = 2048
  return pl.pallas_call(
      _mm_body,
      grid=(n // blk,),
      in_specs=[
          pl.BlockSpec((blk, 320), lambda i: (i, 0)),
          pl.BlockSpec((320, 128), lambda i: (0, 0)),
          pl.BlockSpec((1, 128), lambda i: (0, 0)),
      ],
      out_specs=pl.BlockSpec((blk, 128), lambda i: (i, 0)),
      out_shape=jax.ShapeDtypeStruct((n, 128), jnp.float32),
  )(e, w_pad, b_pad)


@jax.jit
def kernel(x, emb_table, W, b):
  idx = x.reshape(_NW, _NCHUNK, _CHUNK)
  gathered = _gather(idx, emb_table)
  e = gathered.reshape(x.shape[0], _D * x.shape[1])
  w_pad = jnp.zeros((320, 128), jnp.float32).at[:, :9].set(W)
  b_pad = jnp.zeros((1, 128), jnp.float32).at[0, :9].set(b)
  out = _matmul(e, w_pad, b_pad)
  return out[:, :9]


# speedup vs baseline: 1.0834x; 1.0834x over previous
import functools

import jax
import jax.numpy as jnp
from jax import lax
from jax.experimental import pallas as pl
from jax.experimental.pallas import tpu as pltpu
from jax.experimental.pallas import tpu_sc as plsc

_NC = 2
_NS = 16
_NW = _NC * _NS

_V = 1000000
_B = 16384 * 5
_N = 16384
_CHUNK = 128
_PER_W = _B // _NW
_NCHUNK = _PER_W // _CHUNK

_TBLK = 2048


def _transform_body(t_ref, o_ref):
  t = jnp.swapaxes(t_ref[...], 0, 1)
  t3 = t.reshape(_TBLK // 2, 2, 64)
  o_ref[...] = jnp.concatenate([t3[:, 0, :], t3[:, 1, :]], axis=1)


def _transform(tt):
  n_blk = pl.cdiv(_V, _TBLK)
  return pl.pallas_call(
      _transform_body,
      grid=(n_blk,),
      in_specs=[pl.BlockSpec((64, _TBLK), lambda i: (0, i))],
      out_specs=pl.BlockSpec((_TBLK // 2, 128), lambda i: (i, 0)),
      out_shape=jax.ShapeDtypeStruct((_V // 2, 128), jnp.float32),
  )(tt)


def _make_gather():
  mesh = plsc.VectorSubcoreMesh(core_axis_name="c", subcore_axis_name="s")

  @functools.partial(
      pl.kernel,
      mesh=mesh,
      out_type=jax.ShapeDtypeStruct((_B, 128), jnp.float32),
      scratch_types=[
          pltpu.VMEM((_NCHUNK, _CHUNK), jnp.int32),
          pltpu.VMEM((_CHUNK, 128), jnp.float32),
          pltpu.VMEM((_CHUNK, 128), jnp.float32),
          pltpu.SemaphoreType.DMA,
          pltpu.SemaphoreType.DMA,
      ],
  )
  def gather_k(idx_hbm, table_hbm, out_hbm, idx_v, rows0, rows1, sem0, sem1):
    wid = lax.axis_index("s") * _NC + lax.axis_index("c")
    base = wid * _PER_W
    pltpu.sync_copy(idx_hbm.at[wid], idx_v)

    rows = (rows0, rows1)
    sems = (sem0, sem1)
    pltpu.async_copy(table_hbm.at[idx_v.at[0]], rows0, sem0)
    for j in range(_NCHUNK):
      if j + 1 < _NCHUNK:
        nxt = (j + 1) % 2
        pltpu.async_copy(table_hbm.at[idx_v.at[j + 1]], rows[nxt], sems[nxt])
      pltpu.make_async_copy(
          table_hbm.at[idx_v.at[j]], rows[j % 2], sems[j % 2]
      ).wait()
      pltpu.sync_copy(rows[j % 2], out_hbm.at[pl.ds(base + j * _CHUNK, _CHUNK)])

  return gather_k


_gather = _make_gather()

_MBLK = 2048


def _mm_body(g_ref, x_ref, w_ref, b_ref, o_ref):
  w = pl.program_id(1)
  v = x_ref[0, 0, :]
  odd = (v & 1).reshape(_MBLK, 1) == 1
  lane = lax.broadcasted_iota(jnp.int32, (_MBLK, 128), 1) >= 64
  g2 = jnp.where(lane == odd, g_ref[...], 0.0)
  acc = jnp.dot(g2, w_ref[...], preferred_element_type=jnp.float32)

  @pl.when(w == 0)
  def _():
    o_ref[...] = acc + b_ref[...]

  @pl.when(w != 0)
  def _():
    o_ref[...] += acc


def _matmul(g, xt3, w2, b_pad):
  nblk = _N // _MBLK
  return pl.pallas_call(
      _mm_body,
      grid=(nblk, 5),
      in_specs=[
          pl.BlockSpec((_MBLK, 128), lambda i, w: (w * (_N // _MBLK) + i, 0)),
          pl.BlockSpec((1, 1, _MBLK), lambda i, w: (w, 0, i)),
          pl.BlockSpec((128, 128), lambda i, w: (w, 0)),
          pl.BlockSpec((1, 128), lambda i, w: (0, 0)),
      ],
      out_specs=pl.BlockSpec((_MBLK, 128), lambda i, w: (i, 0)),
      out_shape=jax.ShapeDtypeStruct((_N, 128), jnp.float32),
  )(g, xt3, w2, b_pad)


@jax.jit
def kernel(x, emb_table, W, b):
  xt = x.T
  idx = (xt.reshape(-1) >> 1).reshape(_NW, _NCHUNK, _CHUNK)
  j_table = _transform(emb_table.T)
  g = _gather(idx, j_table)
  wr = W.reshape(5, 64, 9)
  w2 = jnp.pad(
      jnp.concatenate([wr, wr], axis=1), ((0, 0), (0, 0), (0, 119))
  ).reshape(640, 128)
  b_pad = jnp.zeros((1, 128), jnp.float32).at[0, :9].set(b)
  out = _matmul(g, xt.reshape(5, 1, _N), w2, b_pad)
  return out[:, :9]
